# baseline (speedup 1.0000x reference)
import functools

import jax
import jax.numpy as jnp
from jax import lax
from jax.experimental import pallas as pl
from jax.experimental.pallas import tpu as pltpu
from jax.experimental.pallas import tpu_sc as plsc

_BIG_I32 = 2**30
_NEG_INF = float("-inf")
_POS_INF = float("inf")


_XN = 32768
_XROWS_W = 4
_DN = 1024
_DROWS = 16384
_DROWS_W = _DROWS // 32
_DCH = 32
_DNCH = _DROWS_W // _DCH


def _lane_iota():
    return lax.broadcasted_iota(jnp.int32, (16,), 0)


def _allmax(v):
    lane = _lane_iota()
    for s in (8, 4, 2, 1):
        perm = jnp.bitwise_xor(lane, s)
        v = jnp.maximum(v, v.at[perm].get(mode="promise_in_bounds"))
    return v


def _merge_rounds(vs, ts, k, accv, acci):
    lane = _lane_iota()
    neg = jnp.full((16,), _NEG_INF, jnp.float32)
    negi = jnp.full((16,), -_BIG_I32, jnp.int32)
    vs = list(vs)
    ts = list(ts)
    for r in range(k):
        m = _allmax(vs[0])
        gidx = ts[0] * 16 + lane
        j = -_allmax(jnp.where(vs[0] == m, -gidx, negi))
        accv = jnp.where(lane == r, m, accv)
        acci = jnp.where(lane == r, j, acci)
        if r + 1 < k:
            msk = jnp.logical_and(vs[0] == m, gidx == j)
            for q in range(len(vs) - 1):
                vs[q] = jnp.where(msk, vs[q + 1], vs[q])
                ts[q] = jnp.where(msk, ts[q + 1], ts[q])
            vs[-1] = jnp.where(msk, neg, vs[-1])
    return accv, acci


_ZROWS = 4096
_ZROWS_W = _ZROWS // 32
_ZCH = 16
_ZNCH = _ZROWS_W // _ZCH


def _sc_body(x_hbm, z_hbm, d_hbm, xv_hbm, xi_hbm, zv_hbm, zi_hbm,
             dv_hbm, di_hbm,
             row_v, db0_v, db1_v, xo_v, xo_i, zo_v, zo_i, do_v, do_i,
             sem0, sem1):
    wid = lax.axis_index("c") * 16 + lax.axis_index("s")
    neg = jnp.full((16,), _NEG_INF, jnp.float32)
    zero = jnp.zeros((16,), jnp.int32)

    for rr in range(_XROWS_W):
        row = wid * _XROWS_W + rr
        pltpu.sync_copy(x_hbm.at[row], row_v)

        def xstep(i, st):
            v1, v2, v3, v4, t1, t2, t3, t4 = st
            cur = row_v[pl.ds(i * 16, 16)]
            ti = jnp.full((16,), i, jnp.int32)
            gt1 = cur > v1
            gt2 = cur > v2
            gt3 = cur > v3
            gt4 = cur > v4
            v4 = jnp.where(gt3, v3, jnp.where(gt4, cur, v4))
            t4 = jnp.where(gt3, t3, jnp.where(gt4, ti, t4))
            v3 = jnp.where(gt2, v2, jnp.where(gt3, cur, v3))
            t3 = jnp.where(gt2, t2, jnp.where(gt3, ti, t3))
            v2 = jnp.where(gt1, v1, jnp.where(gt2, cur, v2))
            t2 = jnp.where(gt1, t1, jnp.where(gt2, ti, t2))
            v1 = jnp.where(gt1, cur, v1)
            t1 = jnp.where(gt1, ti, t1)
            return (v1, v2, v3, v4, t1, t2, t3, t4)

        v1, v2, v3, v4, t1, t2, t3, t4 = lax.fori_loop(
            0, _XN // 16, xstep,
            (neg, neg, neg, neg, zero, zero, zero, zero), unroll=8)

        accv, acci = _merge_rounds(
            (v1, v2, v3, v4), (t1, t2, t3, t4), 4,
            jnp.zeros((16,), jnp.float32), jnp.zeros((16,), jnp.int32))
        xo_v[rr] = accv
        xo_i[rr] = acci

    xbase = wid * _XROWS_W
    pltpu.sync_copy(xo_v, xv_hbm.at[pl.ds(xbase, _XROWS_W)])
    pltpu.sync_copy(xo_i, xi_hbm.at[pl.ds(xbase, _XROWS_W)])

    dbase = wid * _DROWS_W
    bufs = (db0_v, db1_v)
    sems = (sem0, sem1)
    copies = [None, None]
    copies[0] = pltpu.async_copy(
        d_hbm.at[pl.ds(dbase, _DCH)], bufs[0], sems[0])
    for c in range(_DNCH):
        buf = bufs[c % 2]
        if c + 1 < _DNCH:
            copies[(c + 1) % 2] = pltpu.async_copy(
                d_hbm.at[pl.ds(dbase + (c + 1) * _DCH, _DCH)],
                bufs[(c + 1) % 2], sems[(c + 1) % 2])
        copies[c % 2].wait()

        def rowbody(rr, _):
            def dstep(i, st):
                v1, v2, t1, t2 = st
                cur = buf[rr, pl.ds(i * 16, 16)]
                ti = jnp.full((16,), i, jnp.int32)
                gt1 = cur > v1
                gt2 = cur > v2
                v2n = jnp.where(gt1, v1, jnp.where(gt2, cur, v2))
                t2n = jnp.where(gt1, t1, jnp.where(gt2, ti, t2))
                v1n = jnp.where(gt1, cur, v1)
                t1n = jnp.where(gt1, ti, t1)
                return (v1n, v2n, t1n, t2n)

            v1, v2, t1, t2 = lax.fori_loop(
                0, _DN // 16, dstep, (neg, neg, zero, zero), unroll=8)
            accv, acci = _merge_rounds(
                (v1, v2), (t1, t2), 2,
                jnp.zeros((16,), jnp.float32), jnp.zeros((16,), jnp.int32))
            do_v[rr] = accv
            do_i[rr] = acci
            return 0

        lax.fori_loop(0, _DCH, rowbody, 0)
        pltpu.sync_copy(do_v, dv_hbm.at[pl.ds(dbase + c * _DCH, _DCH)])
        pltpu.sync_copy(do_i, di_hbm.at[pl.ds(dbase + c * _DCH, _DCH)])

    zbase = wid * _ZROWS_W
    copies[0] = pltpu.async_copy(
        z_hbm.at[pl.ds(zbase * 2, _ZCH * 2)], bufs[0], sems[0])
    for c in range(_ZNCH):
        buf = bufs[c % 2]
        if c + 1 < _ZNCH:
            copies[(c + 1) % 2] = pltpu.async_copy(
                z_hbm.at[pl.ds((zbase + (c + 1) * _ZCH) * 2, _ZCH * 2)],
                bufs[(c + 1) % 2], sems[(c + 1) % 2])
        copies[c % 2].wait()

        def zrowbody(rr, _):
            def zstep_a(i, st):
                v1, t1 = st
                cur = buf[2 * rr, pl.ds(i * 16, 16)]
                gt = cur > v1
                return (jnp.where(gt, cur, v1),
                        jnp.where(gt, jnp.full((16,), i, jnp.int32), t1))

            def zstep_b(i, st):
                v1, t1 = st
                cur = buf[2 * rr + 1, pl.ds((i - 64) * 16, 16)]
                gt = cur > v1
                return (jnp.where(gt, cur, v1),
                        jnp.where(gt, jnp.full((16,), i, jnp.int32), t1))

            st = lax.fori_loop(0, _DN // 16, zstep_a, (neg, zero), unroll=8)
            v1, t1 = lax.fori_loop(_DN // 16, 2 * (_DN // 16), zstep_b, st,
                                   unroll=8)
            accv, acci = _merge_rounds(
                (v1,), (t1,), 1,
                jnp.zeros((16,), jnp.float32), jnp.zeros((16,), jnp.int32))
            zo_v[rr] = accv
            zo_i[rr] = acci
            return 0

        lax.fori_loop(0, _ZCH, zrowbody, 0)
        pltpu.sync_copy(zo_v, zv_hbm.at[pl.ds(zbase + c * _ZCH, _ZCH)])
        pltpu.sync_copy(zo_i, zi_hbm.at[pl.ds(zbase + c * _ZCH, _ZCH)])


def _topk_sc(x, zhalf, dflat):
    xrows = x.shape[0]
    mesh = plsc.VectorSubcoreMesh(core_axis_name="c", subcore_axis_name="s")
    kern = pl.kernel(
        _sc_body,
        out_type=[
            jax.ShapeDtypeStruct((xrows, 16), jnp.float32),
            jax.ShapeDtypeStruct((xrows, 16), jnp.int32),
            jax.ShapeDtypeStruct((_ZROWS, 16), jnp.float32),
            jax.ShapeDtypeStruct((_ZROWS, 16), jnp.int32),
            jax.ShapeDtypeStruct((_DROWS, 16), jnp.float32),
            jax.ShapeDtypeStruct((_DROWS, 16), jnp.int32),
        ],
        mesh=mesh,
        scratch_types=[
            pltpu.VMEM((_XN,), jnp.float32),
            pltpu.VMEM((_DCH, _DN), jnp.float32),
            pltpu.VMEM((_DCH, _DN), jnp.float32),
            pltpu.VMEM((_XROWS_W, 16), jnp.float32),
            pltpu.VMEM((_XROWS_W, 16), jnp.int32),
            pltpu.VMEM((_ZCH, 16), jnp.float32),
            pltpu.VMEM((_ZCH, 16), jnp.int32),
            pltpu.VMEM((_DCH, 16), jnp.float32),
            pltpu.VMEM((_DCH, 16), jnp.int32),
            pltpu.SemaphoreType.DMA,
            pltpu.SemaphoreType.DMA,
        ],
    )
    xv, xi, zv, zi, dv, di = kern(x, zhalf, dflat)
    return (xv[:, :4], xi[:, :4], zv[:, :1], zi[:, :1],
            dv[:, :2], di[:, :2])


_YB, _ZW, _DB = 256, 512, 8
_YS, _ZS, _DS = 16, 4, 16
_Z0, _D0 = _YS, _YS + _ZS
_STEPS = _YS + _ZS + _DS


def _topk_axis(blk, k, axis, largest=True, idx_base=0):
    idx = jax.lax.broadcasted_iota(jnp.int32, blk.shape, axis) + idx_base
    fill = _NEG_INF if largest else _POS_INF
    cur = blk
    vs, js = [], []
    for t in range(k):
        if largest:
            m = jnp.max(cur, axis=axis, keepdims=True)
        else:
            m = jnp.min(cur, axis=axis, keepdims=True)
        j = jnp.min(jnp.where(cur == m, idx, _BIG_I32), axis=axis, keepdims=True)
        vs.append(m)
        js.append(j)
        if t + 1 < k:
            cur = jnp.where(idx == j, fill, cur)
    return vs, js


def _body(y_ref, z_ref, d_ref,
          y1_ref, i1_ref, y2_ref, i2_ref,
          i3_ref, i4_ref,
          d2_ref, i8_ref):
    g = pl.program_id(0)

    @pl.when(g < _Z0)
    def _y_phase():
        gy = g
        blk = y_ref[...]

        (m1, m2), (j1, j2) = _topk_axis(blk, 2, 0, largest=True,
                                        idx_base=gy * _YB)

        @pl.when(gy == 0)
        def _():
            y1_ref[...] = jnp.concatenate([m1, m2], axis=0)
            i1_ref[...] = jnp.concatenate([j1, j2], axis=0)

        @pl.when(gy > 0)
        def _():
            v1 = y1_ref[0:1, :]
            v2 = y1_ref[1:2, :]
            p1 = i1_ref[0:1, :]
            p2 = i1_ref[1:2, :]
            take1 = m1 > v1
            a_v = jnp.where(take1, v1, v2)
            a_i = jnp.where(take1, p1, p2)
            b_v = jnp.where(take1, m2, m1)
            b_i = jnp.where(take1, j2, j1)
            take2 = b_v > a_v
            y1_ref[...] = jnp.concatenate(
                [jnp.where(take1, m1, v1), jnp.where(take2, b_v, a_v)], axis=0)
            i1_ref[...] = jnp.concatenate(
                [jnp.where(take1, j1, p1), jnp.where(take2, b_i, a_i)], axis=0)

        (n1, n2), (k1, k2) = _topk_axis(blk, 2, 1, largest=False)
        y2_ref[...] = jnp.concatenate([n1, n2], axis=1)
        i2_ref[...] = jnp.concatenate([k1, k2], axis=1)

    @pl.when(jnp.logical_and(g >= _Z0, g < _D0))
    def _z_phase():
        gz = g - _Z0
        blk = z_ref[...]

        _, js = _topk_axis(blk, 2, 0, largest=True)
        i3_ref[...] = jnp.concatenate(js, axis=0)

        _, js = _topk_axis(blk, 3, 1, largest=True)
        i4_ref[...] = jnp.concatenate(js, axis=1)

    @pl.when(g >= _D0)
    def _d_phase():
        blk = d_ref[...]

        vs, js = _topk_axis(blk, 2, 1, largest=True)
        d2_ref[...] = jnp.concatenate(vs, axis=1)
        i8_ref[...] = jnp.concatenate(js, axis=1)


def _fused(y, z, d):
    yr, yn = y.shape
    a0, a1, a2 = z.shape
    db, d1, d2n = d.shape

    def ym(g):
        return (jnp.clip(g, 0, _YS - 1), 0)

    def zm(g):
        return (0, 0, jnp.clip(g - _Z0, 0, _ZS - 1))

    def dm(g):
        return (jnp.clip(g - _D0, 0, _DS - 1), 0, 0)

    return pl.pallas_call(
        _body,
        grid=(_STEPS,),
        in_specs=[
            pl.BlockSpec((_YB, yn), ym),
            pl.BlockSpec((a0, a1, _ZW), zm),
            pl.BlockSpec((_DB, d1, d2n), dm),
        ],
        out_specs=[
            pl.BlockSpec((2, yn), lambda g: (0, 0)),
            pl.BlockSpec((2, yn), lambda g: (0, 0)),
            pl.BlockSpec((_YB, 2), ym),
            pl.BlockSpec((_YB, 2), ym),
            pl.BlockSpec((2, a1, _ZW), zm),
            pl.BlockSpec((a0, 3, _ZW), zm),
            pl.BlockSpec((_DB, 2, d2n), dm),
            pl.BlockSpec((_DB, 2, d2n), dm),
        ],
        out_shape=[
            jax.ShapeDtypeStruct((2, yn), jnp.float32),
            jax.ShapeDtypeStruct((2, yn), jnp.int32),
            jax.ShapeDtypeStruct((yr, 2), jnp.float32),
            jax.ShapeDtypeStruct((yr, 2), jnp.int32),
            jax.ShapeDtypeStruct((2, a1, a2), jnp.int32),
            jax.ShapeDtypeStruct((a0, 3, a2), jnp.int32),
            jax.ShapeDtypeStruct((db, 2, d2n), jnp.float32),
            jax.ShapeDtypeStruct((db, 2, d2n), jnp.int32),
        ],
    )(y, z, d)


def kernel(x, y, z, d):
    b0, b1, a1, a2 = d.shape
    za, zb, zc = z.shape
    x0, i0, z1, i5, d3, i9 = _topk_sc(
        x, z.reshape(za * zb * 2, zc // 2), d.reshape(b0 * b1 * a1, a2))
    z1 = z1.reshape(za, zb, 1)
    i5 = i5.reshape(za, zb, 1)

    (y1, i1, y2, i2, i3, i4,
     d2, i8) = _fused(y, z, d.reshape(b0 * b1, a1, a2))
    d2 = d2.reshape(b0, b1, 2, a2)
    i8 = i8.reshape(b0, b1, 2, a2)
    d3 = d3.reshape(b0, b1, a1, 2)
    i9 = i9.reshape(b0, b1, a1, 2)
    return (x0, y1, y2, i0, i1, i2, z1, i3, i4, i5, d2, d3, i8, i9)


# speedup vs baseline: 1.0596x; 1.0596x over previous
import functools

import jax
import jax.numpy as jnp
from jax import lax
from jax.experimental import pallas as pl
from jax.experimental.pallas import tpu as pltpu
from jax.experimental.pallas import tpu_sc as plsc

_BIG_I32 = 2**30
_NEG_INF = float("-inf")
_POS_INF = float("inf")


_XN = 32768
_XROWS_W = 4
_DN = 1024
_DROWS = 16384
_DNCH = 11
_DCH = 32
_DROWS_W = _DNCH * _DCH
_DF = _DROWS_W * 32


def _lane_iota():
    return lax.broadcasted_iota(jnp.int32, (16,), 0)


def _allmax(v):
    lane = _lane_iota()
    for s in (8, 4, 2, 1):
        perm = jnp.bitwise_xor(lane, s)
        v = jnp.maximum(v, v.at[perm].get(mode="promise_in_bounds"))
    return v


def _merge_rounds(vs, ts, k, accv, acci):
    lane = _lane_iota()
    neg = jnp.full((16,), _NEG_INF, jnp.float32)
    negi = jnp.full((16,), -_BIG_I32, jnp.int32)
    vs = list(vs)
    ts = list(ts)
    for r in range(k):
        m = _allmax(vs[0])
        gidx = ts[0] * 16 + lane
        j = -_allmax(jnp.where(vs[0] == m, -gidx, negi))
        accv = jnp.where(lane == r, m, accv)
        acci = jnp.where(lane == r, j, acci)
        if r + 1 < k:
            msk = jnp.logical_and(vs[0] == m, gidx == j)
            for q in range(len(vs) - 1):
                vs[q] = jnp.where(msk, vs[q + 1], vs[q])
                ts[q] = jnp.where(msk, ts[q + 1], ts[q])
            vs[-1] = jnp.where(msk, neg, vs[-1])
    return accv, acci


def _sc_body(x_hbm, d_hbm, xv_hbm, xi_hbm,
             dv_hbm, di_hbm,
             row_v, db0_v, db1_v, xo_v, xo_i, do_v, do_i,
             sem0, sem1):
    wid = lax.axis_index("c") * 16 + lax.axis_index("s")
    neg = jnp.full((16,), _NEG_INF, jnp.float32)
    zero = jnp.zeros((16,), jnp.int32)

    for rr in range(_XROWS_W):
        row = wid * _XROWS_W + rr
        pltpu.sync_copy(x_hbm.at[row], row_v)

        def xstep(i, st):
            v1, v2, v3, v4, t1, t2, t3, t4 = st
            cur = row_v[pl.ds(i * 16, 16)]
            ti = jnp.full((16,), i, jnp.int32)
            gt1 = cur > v1
            gt2 = cur > v2
            gt3 = cur > v3
            gt4 = cur > v4
            v4 = jnp.where(gt3, v3, jnp.where(gt4, cur, v4))
            t4 = jnp.where(gt3, t3, jnp.where(gt4, ti, t4))
            v3 = jnp.where(gt2, v2, jnp.where(gt3, cur, v3))
            t3 = jnp.where(gt2, t2, jnp.where(gt3, ti, t3))
            v2 = jnp.where(gt1, v1, jnp.where(gt2, cur, v2))
            t2 = jnp.where(gt1, t1, jnp.where(gt2, ti, t2))
            v1 = jnp.where(gt1, cur, v1)
            t1 = jnp.where(gt1, ti, t1)
            return (v1, v2, v3, v4, t1, t2, t3, t4)

        v1, v2, v3, v4, t1, t2, t3, t4 = lax.fori_loop(
            0, _XN // 16, xstep,
            (neg, neg, neg, neg, zero, zero, zero, zero), unroll=8)

        accv, acci = _merge_rounds(
            (v1, v2, v3, v4), (t1, t2, t3, t4), 4,
            jnp.zeros((16,), jnp.float32), jnp.zeros((16,), jnp.int32))
        xo_v[rr] = accv
        xo_i[rr] = acci

    xbase = wid * _XROWS_W
    pltpu.sync_copy(xo_v, xv_hbm.at[pl.ds(xbase, _XROWS_W)])
    pltpu.sync_copy(xo_i, xi_hbm.at[pl.ds(xbase, _XROWS_W)])

    dbase = wid * _DROWS_W
    bufs = (db0_v, db1_v)
    sems = (sem0, sem1)
    copies = [None, None]
    copies[0] = pltpu.async_copy(
        d_hbm.at[pl.ds(dbase, _DCH)], bufs[0], sems[0])
    for c in range(_DNCH):
        buf = bufs[c % 2]
        if c + 1 < _DNCH:
            copies[(c + 1) % 2] = pltpu.async_copy(
                d_hbm.at[pl.ds(dbase + (c + 1) * _DCH, _DCH)],
                bufs[(c + 1) % 2], sems[(c + 1) % 2])
        copies[c % 2].wait()

        def rowbody(rr, _):
            def dstep(i, st):
                v1, v2, t1, t2 = st
                cur = buf[rr, pl.ds(i * 16, 16)]
                ti = jnp.full((16,), i, jnp.int32)
                gt1 = cur > v1
                gt2 = cur > v2
                v2n = jnp.where(gt1, v1, jnp.where(gt2, cur, v2))
                t2n = jnp.where(gt1, t1, jnp.where(gt2, ti, t2))
                v1n = jnp.where(gt1, cur, v1)
                t1n = jnp.where(gt1, ti, t1)
                return (v1n, v2n, t1n, t2n)

            v1, v2, t1, t2 = lax.fori_loop(
                0, _DN // 16, dstep, (neg, neg, zero, zero), unroll=8)
            accv, acci = _merge_rounds(
                (v1, v2), (t1, t2), 2,
                jnp.zeros((16,), jnp.float32), jnp.zeros((16,), jnp.int32))
            do_v[rr] = accv
            do_i[rr] = acci
            return 0

        lax.fori_loop(0, _DCH, rowbody, 0)
        pltpu.sync_copy(do_v, dv_hbm.at[pl.ds(dbase + c * _DCH, _DCH)])
        pltpu.sync_copy(do_i, di_hbm.at[pl.ds(dbase + c * _DCH, _DCH)])


def _topk_sc(x, dflat):
    xrows = x.shape[0]
    mesh = plsc.VectorSubcoreMesh(core_axis_name="c", subcore_axis_name="s")
    kern = pl.kernel(
        _sc_body,
        out_type=[
            jax.ShapeDtypeStruct((xrows, 16), jnp.float32),
            jax.ShapeDtypeStruct((xrows, 16), jnp.int32),
            jax.ShapeDtypeStruct((_DF, 16), jnp.float32),
            jax.ShapeDtypeStruct((_DF, 16), jnp.int32),
        ],
        mesh=mesh,
        scratch_types=[
            pltpu.VMEM((_XN,), jnp.float32),
            pltpu.VMEM((_DCH, _DN), jnp.float32),
            pltpu.VMEM((_DCH, _DN), jnp.float32),
            pltpu.VMEM((_XROWS_W, 16), jnp.float32),
            pltpu.VMEM((_XROWS_W, 16), jnp.int32),
            pltpu.VMEM((_DCH, 16), jnp.float32),
            pltpu.VMEM((_DCH, 16), jnp.int32),
            pltpu.SemaphoreType.DMA,
            pltpu.SemaphoreType.DMA,
        ],
    )
    xv, xi, dv, di = kern(x, dflat)
    return xv[:, :4], xi[:, :4], dv[:, :2], di[:, :2]


_YB, _ZW, _DB = 256, 512, 8
_YS, _ZS, _DS = 16, 4, 16
_Z0, _D0 = _YS, _YS + _ZS
_STEPS = _YS + _ZS + _DS


def _topk_axis(blk, k, axis, largest=True, idx_base=0):
    idx = jax.lax.broadcasted_iota(jnp.int32, blk.shape, axis) + idx_base
    fill = _NEG_INF if largest else _POS_INF
    cur = blk
    vs, js = [], []
    for t in range(k):
        if largest:
            m = jnp.max(cur, axis=axis, keepdims=True)
        else:
            m = jnp.min(cur, axis=axis, keepdims=True)
        j = jnp.min(jnp.where(cur == m, idx, _BIG_I32), axis=axis, keepdims=True)
        vs.append(m)
        js.append(j)
        if t + 1 < k:
            cur = jnp.where(idx == j, fill, cur)
    return vs, js


_D9S = _DNCH


def _body(y_ref, z_ref, d_ref,
          y1_ref, i1_ref, y2_ref, i2_ref,
          i3_ref, i4_ref, z1_ref, i5_ref,
          d2_ref, i8_ref, d3_ref, i9_ref,
          v_s, p_s):
    g = pl.program_id(0)

    @pl.when(g < _Z0)
    def _y_phase():
        gy = g
        blk = y_ref[...]

        (m1, m2), (j1, j2) = _topk_axis(blk, 2, 0, largest=True,
                                        idx_base=gy * _YB)

        @pl.when(gy == 0)
        def _():
            y1_ref[...] = jnp.concatenate([m1, m2], axis=0)
            i1_ref[...] = jnp.concatenate([j1, j2], axis=0)

        @pl.when(gy > 0)
        def _():
            v1 = y1_ref[0:1, :]
            v2 = y1_ref[1:2, :]
            p1 = i1_ref[0:1, :]
            p2 = i1_ref[1:2, :]
            take1 = m1 > v1
            a_v = jnp.where(take1, v1, v2)
            a_i = jnp.where(take1, p1, p2)
            b_v = jnp.where(take1, m2, m1)
            b_i = jnp.where(take1, j2, j1)
            take2 = b_v > a_v
            y1_ref[...] = jnp.concatenate(
                [jnp.where(take1, m1, v1), jnp.where(take2, b_v, a_v)], axis=0)
            i1_ref[...] = jnp.concatenate(
                [jnp.where(take1, j1, p1), jnp.where(take2, b_i, a_i)], axis=0)

        (n1, n2), (k1, k2) = _topk_axis(blk, 2, 1, largest=False)
        y2_ref[...] = jnp.concatenate([n1, n2], axis=1)
        i2_ref[...] = jnp.concatenate([k1, k2], axis=1)

    @pl.when(jnp.logical_and(g >= _Z0, g < _D0))
    def _z_phase():
        gz = g - _Z0
        blk = z_ref[...]

        _, js = _topk_axis(blk, 2, 0, largest=True)
        i3_ref[...] = jnp.concatenate(js, axis=0)

        _, js = _topk_axis(blk, 3, 1, largest=True)
        i4_ref[...] = jnp.concatenate(js, axis=1)

        lanes = jax.lax.broadcasted_iota(jnp.int32, blk.shape, 2) + gz * _ZW
        lm = jnp.max(blk, axis=2)
        lj = jnp.min(jnp.where(blk == lm[:, :, None], lanes, _BIG_I32), axis=2)

        @pl.when(gz == 0)
        def _():
            v_s[...] = lm
            p_s[...] = lj

        @pl.when(gz > 0)
        def _():
            take = lm > v_s[...]
            v_s[...] = jnp.where(take, lm, v_s[...])
            p_s[...] = jnp.where(take, lj, p_s[...])

        @pl.when(gz == _ZS - 1)
        def _():
            z1_ref[...] = v_s[...]
            i5_ref[...] = p_s[...]

    @pl.when(g >= _D0)
    def _d_phase():
        blk = d_ref[...]

        vs, js = _topk_axis(blk, 2, 1, largest=True)
        d2_ref[...] = jnp.concatenate(vs, axis=1)
        i8_ref[...] = jnp.concatenate(js, axis=1)

        @pl.when(g >= _D0 + _D9S)
        def _():
            vs2, js2 = _topk_axis(blk, 2, 2, largest=True)
            d3_ref[...] = jnp.concatenate(vs2, axis=2)
            i9_ref[...] = jnp.concatenate(js2, axis=2)


def _fused(y, z, d):
    yr, yn = y.shape
    a0, a1, a2 = z.shape
    db, d1, d2n = d.shape

    def ym(g):
        return (jnp.clip(g, 0, _YS - 1), 0)

    def zm(g):
        return (0, 0, jnp.clip(g - _Z0, 0, _ZS - 1))

    def dm(g):
        return (jnp.clip(g - _D0, 0, _DS - 1), 0, 0)

    return pl.pallas_call(
        _body,
        grid=(_STEPS,),
        in_specs=[
            pl.BlockSpec((_YB, yn), ym),
            pl.BlockSpec((a0, a1, _ZW), zm),
            pl.BlockSpec((_DB, d1, d2n), dm),
        ],
        out_specs=[
            pl.BlockSpec((2, yn), lambda g: (0, 0)),
            pl.BlockSpec((2, yn), lambda g: (0, 0)),
            pl.BlockSpec((_YB, 2), ym),
            pl.BlockSpec((_YB, 2), ym),
            pl.BlockSpec((2, a1, _ZW), zm),
            pl.BlockSpec((a0, 3, _ZW), zm),
            pl.BlockSpec((a0, a1), lambda g: (0, 0)),
            pl.BlockSpec((a0, a1), lambda g: (0, 0)),
            pl.BlockSpec((_DB, 2, d2n), dm),
            pl.BlockSpec((_DB, 2, d2n), dm),
            pl.BlockSpec((_DB, d1, 2), dm),
            pl.BlockSpec((_DB, d1, 2), dm),
        ],
        out_shape=[
            jax.ShapeDtypeStruct((2, yn), jnp.float32),
            jax.ShapeDtypeStruct((2, yn), jnp.int32),
            jax.ShapeDtypeStruct((yr, 2), jnp.float32),
            jax.ShapeDtypeStruct((yr, 2), jnp.int32),
            jax.ShapeDtypeStruct((2, a1, a2), jnp.int32),
            jax.ShapeDtypeStruct((a0, 3, a2), jnp.int32),
            jax.ShapeDtypeStruct((a0, a1), jnp.float32),
            jax.ShapeDtypeStruct((a0, a1), jnp.int32),
            jax.ShapeDtypeStruct((db, 2, d2n), jnp.float32),
            jax.ShapeDtypeStruct((db, 2, d2n), jnp.int32),
            jax.ShapeDtypeStruct((db, d1, 2), jnp.float32),
            jax.ShapeDtypeStruct((db, d1, 2), jnp.int32),
        ],
        scratch_shapes=[
            pltpu.VMEM((a0, a1), jnp.float32),
            pltpu.VMEM((a0, a1), jnp.int32),
        ],
    )(y, z, d)


def kernel(x, y, z, d):
    b0, b1, a1, a2 = d.shape
    x0, i0, d3sc, i9sc = _topk_sc(x, d.reshape(b0 * b1 * a1, a2))

    (y1, i1, y2, i2, i3, i4, z1, i5,
     d2, i8, d3tc, i9tc) = _fused(y, z, d.reshape(b0 * b1, a1, a2))
    z1 = z1[:, :, None]
    i5 = i5[:, :, None]
    d2 = d2.reshape(b0, b1, 2, a2)
    i8 = i8.reshape(b0, b1, 2, a2)
    d3 = jnp.concatenate(
        [d3sc, d3tc.reshape(b0 * b1 * a1, 2)[_DF:]], axis=0)
    i9 = jnp.concatenate(
        [i9sc, i9tc.reshape(b0 * b1 * a1, 2)[_DF:]], axis=0)
    d3 = d3.reshape(b0, b1, a1, 2)
    i9 = i9.reshape(b0, b1, a1, 2)
    return (x0, y1, y2, i0, i1, i2, z1, i3, i4, i5, d2, d3, i8, i9)


# speedup vs baseline: 1.1244x; 1.0611x over previous
import functools

import jax
import jax.numpy as jnp
from jax import lax
from jax.experimental import pallas as pl
from jax.experimental.pallas import tpu as pltpu
from jax.experimental.pallas import tpu_sc as plsc

_BIG_I32 = 2**30
_NEG_INF = float("-inf")
_POS_INF = float("inf")


_XN = 32768
_XROWS_W = 4
_DN = 1024
_DROWS = 16384
_DNCH = 16
_DCH = 32
_DROWS_W = _DNCH * _DCH
_DF = _DROWS_W * 32


def _lane_iota():
    return lax.broadcasted_iota(jnp.int32, (16,), 0)


def _allmax(v):
    lane = _lane_iota()
    for s in (8, 4, 2, 1):
        perm = jnp.bitwise_xor(lane, s)
        v = jnp.maximum(v, v.at[perm].get(mode="promise_in_bounds"))
    return v


def _merge_rounds(vs, ts, k, accv, acci):
    lane = _lane_iota()
    neg = jnp.full((16,), _NEG_INF, jnp.float32)
    negi = jnp.full((16,), -_BIG_I32, jnp.int32)
    vs = list(vs)
    ts = list(ts)
    for r in range(k):
        m = _allmax(vs[0])
        gidx = ts[0] * 16 + lane
        j = -_allmax(jnp.where(vs[0] == m, -gidx, negi))
        accv = jnp.where(lane == r, m, accv)
        acci = jnp.where(lane == r, j, acci)
        if r + 1 < k:
            msk = jnp.logical_and(vs[0] == m, gidx == j)
            for q in range(len(vs) - 1):
                vs[q] = jnp.where(msk, vs[q + 1], vs[q])
                ts[q] = jnp.where(msk, ts[q + 1], ts[q])
            vs[-1] = jnp.where(msk, neg, vs[-1])
    return accv, acci


def _sc_body(x_hbm, d_hbm, xv_hbm, xi_hbm,
             dv_hbm, di_hbm,
             row_v, db0_v, db1_v, xo_v, xo_i, do_v, do_i,
             sem0, sem1):
    wid = lax.axis_index("c") * 16 + lax.axis_index("s")
    neg = jnp.full((16,), _NEG_INF, jnp.float32)
    zero = jnp.zeros((16,), jnp.int32)

    for rr in range(_XROWS_W):
        row = wid * _XROWS_W + rr
        pltpu.sync_copy(x_hbm.at[row], row_v)

        def xstep(i, st):
            v1, v2, v3, v4, t1, t2, t3, t4 = st
            cur = row_v[pl.ds(i * 16, 16)]
            ti = jnp.full((16,), i, jnp.int32)
            gt1 = cur > v1
            gt2 = cur > v2
            gt3 = cur > v3
            gt4 = cur > v4
            v4 = jnp.where(gt3, v3, jnp.where(gt4, cur, v4))
            t4 = jnp.where(gt3, t3, jnp.where(gt4, ti, t4))
            v3 = jnp.where(gt2, v2, jnp.where(gt3, cur, v3))
            t3 = jnp.where(gt2, t2, jnp.where(gt3, ti, t3))
            v2 = jnp.where(gt1, v1, jnp.where(gt2, cur, v2))
            t2 = jnp.where(gt1, t1, jnp.where(gt2, ti, t2))
            v1 = jnp.where(gt1, cur, v1)
            t1 = jnp.where(gt1, ti, t1)
            return (v1, v2, v3, v4, t1, t2, t3, t4)

        v1, v2, v3, v4, t1, t2, t3, t4 = lax.fori_loop(
            0, _XN // 16, xstep,
            (neg, neg, neg, neg, zero, zero, zero, zero), unroll=8)

        accv, acci = _merge_rounds(
            (v1, v2, v3, v4), (t1, t2, t3, t4), 4,
            jnp.zeros((16,), jnp.float32), jnp.zeros((16,), jnp.int32))
        xo_v[rr] = accv
        xo_i[rr] = acci

    xbase = wid * _XROWS_W
    pltpu.sync_copy(xo_v, xv_hbm.at[pl.ds(xbase, _XROWS_W)])
    pltpu.sync_copy(xo_i, xi_hbm.at[pl.ds(xbase, _XROWS_W)])

    dbase = wid * _DROWS_W
    bufs = (db0_v, db1_v)
    sems = (sem0, sem1)
    copies = [None, None]
    copies[0] = pltpu.async_copy(
        d_hbm.at[pl.ds(dbase, _DCH)], bufs[0], sems[0])
    for c in range(_DNCH):
        buf = bufs[c % 2]
        if c + 1 < _DNCH:
            copies[(c + 1) % 2] = pltpu.async_copy(
                d_hbm.at[pl.ds(dbase + (c + 1) * _DCH, _DCH)],
                bufs[(c + 1) % 2], sems[(c + 1) % 2])
        copies[c % 2].wait()

        def rowbody(rr, _):
            def dstep(i, st):
                v1, v2, t1, t2 = st
                cur = buf[rr, pl.ds(i * 16, 16)]
                ti = jnp.full((16,), i, jnp.int32)
                gt1 = cur > v1
                gt2 = cur > v2
                v2n = jnp.where(gt1, v1, jnp.where(gt2, cur, v2))
                t2n = jnp.where(gt1, t1, jnp.where(gt2, ti, t2))
                v1n = jnp.where(gt1, cur, v1)
                t1n = jnp.where(gt1, ti, t1)
                return (v1n, v2n, t1n, t2n)

            v1, v2, t1, t2 = lax.fori_loop(
                0, _DN // 16, dstep, (neg, neg, zero, zero), unroll=8)
            accv, acci = _merge_rounds(
                (v1, v2), (t1, t2), 2,
                jnp.zeros((16,), jnp.float32), jnp.zeros((16,), jnp.int32))
            do_v[rr] = accv
            do_i[rr] = acci
            return 0

        lax.fori_loop(0, _DCH, rowbody, 0)
        pltpu.sync_copy(do_v, dv_hbm.at[pl.ds(dbase + c * _DCH, _DCH)])
        pltpu.sync_copy(do_i, di_hbm.at[pl.ds(dbase + c * _DCH, _DCH)])


def _topk_sc(x, dflat):
    xrows = x.shape[0]
    mesh = plsc.VectorSubcoreMesh(core_axis_name="c", subcore_axis_name="s")
    kern = pl.kernel(
        _sc_body,
        out_type=[
            jax.ShapeDtypeStruct((xrows, 16), jnp.float32),
            jax.ShapeDtypeStruct((xrows, 16), jnp.int32),
            jax.ShapeDtypeStruct((_DF, 16), jnp.float32),
            jax.ShapeDtypeStruct((_DF, 16), jnp.int32),
        ],
        mesh=mesh,
        scratch_types=[
            pltpu.VMEM((_XN,), jnp.float32),
            pltpu.VMEM((_DCH, _DN), jnp.float32),
            pltpu.VMEM((_DCH, _DN), jnp.float32),
            pltpu.VMEM((_XROWS_W, 16), jnp.float32),
            pltpu.VMEM((_XROWS_W, 16), jnp.int32),
            pltpu.VMEM((_DCH, 16), jnp.float32),
            pltpu.VMEM((_DCH, 16), jnp.int32),
            pltpu.SemaphoreType.DMA,
            pltpu.SemaphoreType.DMA,
        ],
    )
    xv, xi, dv, di = kern(x, dflat)
    return xv[:, :4], xi[:, :4], dv[:, :2], di[:, :2]


_YB, _ZW, _DB = 256, 512, 8
_YS, _ZS, _DS = 16, 4, 16
_Z0, _D0 = _YS, _YS + _ZS
_STEPS = _YS + _ZS + _DS


def _topk_axis(blk, k, axis, largest=True, idx_base=0):
    idx = jax.lax.broadcasted_iota(jnp.int32, blk.shape, axis) + idx_base
    fill = _NEG_INF if largest else _POS_INF
    cur = blk
    vs, js = [], []
    for t in range(k):
        if largest:
            m = jnp.max(cur, axis=axis, keepdims=True)
        else:
            m = jnp.min(cur, axis=axis, keepdims=True)
        j = jnp.min(jnp.where(cur == m, idx, _BIG_I32), axis=axis, keepdims=True)
        vs.append(m)
        js.append(j)
        if t + 1 < k:
            cur = jnp.where(idx == j, fill, cur)
    return vs, js


_D9S = _DNCH


def _body(y_ref, z_ref, d_ref,
          y1_ref, i1_ref, y2_ref, i2_ref,
          i3_ref, i4_ref, z1_ref, i5_ref,
          d2_ref, i8_ref, d3_ref, i9_ref,
          v_s, p_s):
    g = pl.program_id(0)

    @pl.when(g < _Z0)
    def _y_phase():
        gy = g
        blk = y_ref[...]

        (m1, m2), (j1, j2) = _topk_axis(blk, 2, 0, largest=True,
                                        idx_base=gy * _YB)

        @pl.when(gy == 0)
        def _():
            y1_ref[...] = jnp.concatenate([m1, m2], axis=0)
            i1_ref[...] = jnp.concatenate([j1, j2], axis=0)

        @pl.when(gy > 0)
        def _():
            v1 = y1_ref[0:1, :]
            v2 = y1_ref[1:2, :]
            p1 = i1_ref[0:1, :]
            p2 = i1_ref[1:2, :]
            take1 = m1 > v1
            a_v = jnp.where(take1, v1, v2)
            a_i = jnp.where(take1, p1, p2)
            b_v = jnp.where(take1, m2, m1)
            b_i = jnp.where(take1, j2, j1)
            take2 = b_v > a_v
            y1_ref[...] = jnp.concatenate(
                [jnp.where(take1, m1, v1), jnp.where(take2, b_v, a_v)], axis=0)
            i1_ref[...] = jnp.concatenate(
                [jnp.where(take1, j1, p1), jnp.where(take2, b_i, a_i)], axis=0)

        (n1, n2), (k1, k2) = _topk_axis(blk, 2, 1, largest=False)
        y2_ref[...] = jnp.concatenate([n1, n2], axis=1)
        i2_ref[...] = jnp.concatenate([k1, k2], axis=1)

    @pl.when(jnp.logical_and(g >= _Z0, g < _D0))
    def _z_phase():
        gz = g - _Z0
        blk = z_ref[...]

        _, js = _topk_axis(blk, 2, 0, largest=True)
        i3_ref[...] = jnp.concatenate(js, axis=0)

        _, js = _topk_axis(blk, 3, 1, largest=True)
        i4_ref[...] = jnp.concatenate(js, axis=1)

        lanes = jax.lax.broadcasted_iota(jnp.int32, blk.shape, 2) + gz * _ZW
        lm = jnp.max(blk, axis=2)
        lj = jnp.min(jnp.where(blk == lm[:, :, None], lanes, _BIG_I32), axis=2)

        @pl.when(gz == 0)
        def _():
            v_s[...] = lm
            p_s[...] = lj

        @pl.when(gz > 0)
        def _():
            take = lm > v_s[...]
            v_s[...] = jnp.where(take, lm, v_s[...])
            p_s[...] = jnp.where(take, lj, p_s[...])

        @pl.when(gz == _ZS - 1)
        def _():
            z1_ref[...] = v_s[...]
            i5_ref[...] = p_s[...]

    @pl.when(g >= _D0)
    def _d_phase():
        blk = d_ref[...]

        vs, js = _topk_axis(blk, 2, 1, largest=True)
        d2_ref[...] = jnp.concatenate(vs, axis=1)
        i8_ref[...] = jnp.concatenate(js, axis=1)

        @pl.when(g >= _D0 + _D9S)
        def _():
            vs2, js2 = _topk_axis(blk, 2, 2, largest=True)
            d3_ref[...] = jnp.concatenate(vs2, axis=2)
            i9_ref[...] = jnp.concatenate(js2, axis=2)


def _fused(y, z, d):
    yr, yn = y.shape
    a0, a1, a2 = z.shape
    db, d1, d2n = d.shape

    def ym(g):
        return (jnp.clip(g, 0, _YS - 1), 0)

    def zm(g):
        return (0, 0, jnp.clip(g - _Z0, 0, _ZS - 1))

    def dm(g):
        return (jnp.clip(g - _D0, 0, _DS - 1), 0, 0)

    return pl.pallas_call(
        _body,
        grid=(_STEPS,),
        in_specs=[
            pl.BlockSpec((_YB, yn), ym),
            pl.BlockSpec((a0, a1, _ZW), zm),
            pl.BlockSpec((_DB, d1, d2n), dm),
        ],
        out_specs=[
            pl.BlockSpec((2, yn), lambda g: (0, 0)),
            pl.BlockSpec((2, yn), lambda g: (0, 0)),
            pl.BlockSpec((_YB, 2), ym),
            pl.BlockSpec((_YB, 2), ym),
            pl.BlockSpec((2, a1, _ZW), zm),
            pl.BlockSpec((a0, 3, _ZW), zm),
            pl.BlockSpec((a0, a1), lambda g: (0, 0)),
            pl.BlockSpec((a0, a1), lambda g: (0, 0)),
            pl.BlockSpec((_DB, 2, d2n), dm),
            pl.BlockSpec((_DB, 2, d2n), dm),
            pl.BlockSpec((_DB, d1, 2), dm),
            pl.BlockSpec((_DB, d1, 2), dm),
        ],
        out_shape=[
            jax.ShapeDtypeStruct((2, yn), jnp.float32),
            jax.ShapeDtypeStruct((2, yn), jnp.int32),
            jax.ShapeDtypeStruct((yr, 2), jnp.float32),
            jax.ShapeDtypeStruct((yr, 2), jnp.int32),
            jax.ShapeDtypeStruct((2, a1, a2), jnp.int32),
            jax.ShapeDtypeStruct((a0, 3, a2), jnp.int32),
            jax.ShapeDtypeStruct((a0, a1), jnp.float32),
            jax.ShapeDtypeStruct((a0, a1), jnp.int32),
            jax.ShapeDtypeStruct((db, 2, d2n), jnp.float32),
            jax.ShapeDtypeStruct((db, 2, d2n), jnp.int32),
            jax.ShapeDtypeStruct((db, d1, 2), jnp.float32),
            jax.ShapeDtypeStruct((db, d1, 2), jnp.int32),
        ],
        scratch_shapes=[
            pltpu.VMEM((a0, a1), jnp.float32),
            pltpu.VMEM((a0, a1), jnp.int32),
        ],
    )(y, z, d)


def kernel(x, y, z, d):
    b0, b1, a1, a2 = d.shape
    x0, i0, d3sc, i9sc = _topk_sc(x, d.reshape(b0 * b1 * a1, a2))

    (y1, i1, y2, i2, i3, i4, z1, i5,
     d2, i8, d3tc, i9tc) = _fused(y, z, d.reshape(b0 * b1, a1, a2))
    z1 = z1[:, :, None]
    i5 = i5[:, :, None]
    d2 = d2.reshape(b0, b1, 2, a2)
    i8 = i8.reshape(b0, b1, 2, a2)
    d3 = jnp.concatenate(
        [d3sc, d3tc.reshape(b0 * b1 * a1, 2)[_DF:]], axis=0)
    i9 = jnp.concatenate(
        [i9sc, i9tc.reshape(b0 * b1 * a1, 2)[_DF:]], axis=0)
    d3 = d3.reshape(b0, b1, a1, 2)
    i9 = i9.reshape(b0, b1, a1, 2)
    return (x0, y1, y2, i0, i1, i2, z1, i3, i4, i5, d2, d3, i8, i9)


# speedup vs baseline: 1.1381x; 1.0122x over previous
import functools

import jax
import jax.numpy as jnp
from jax import lax
from jax.experimental import pallas as pl
from jax.experimental.pallas import tpu as pltpu
from jax.experimental.pallas import tpu_sc as plsc

_BIG_I32 = 2**30
_NEG_INF = float("-inf")
_POS_INF = float("inf")


_XN = 32768
_XROWS_W = 4
_DN = 1024
_DROWS = 16384
_DNCH = 16
_DCH = 32
_DROWS_W = _DNCH * _DCH
_DF = _DROWS_W * 32


def _lane_iota():
    return lax.broadcasted_iota(jnp.int32, (16,), 0)


def _allmax(v):
    lane = _lane_iota()
    for s in (8, 4, 2, 1):
        perm = jnp.bitwise_xor(lane, s)
        v = jnp.maximum(v, v.at[perm].get(mode="promise_in_bounds"))
    return v


def _merge_rounds(vs, ts, k, accv, acci):
    lane = _lane_iota()
    neg = jnp.full((16,), _NEG_INF, jnp.float32)
    negi = jnp.full((16,), -_BIG_I32, jnp.int32)
    vs = list(vs)
    ts = list(ts)
    for r in range(k):
        m = _allmax(vs[0])
        gidx = ts[0] * 16 + lane
        j = -_allmax(jnp.where(vs[0] == m, -gidx, negi))
        accv = jnp.where(lane == r, m, accv)
        acci = jnp.where(lane == r, j, acci)
        if r + 1 < k:
            msk = jnp.logical_and(vs[0] == m, gidx == j)
            for q in range(len(vs) - 1):
                vs[q] = jnp.where(msk, vs[q + 1], vs[q])
                ts[q] = jnp.where(msk, ts[q + 1], ts[q])
            vs[-1] = jnp.where(msk, neg, vs[-1])
    return accv, acci


def _sc_body(x_hbm, d_hbm, xv_hbm, xi_hbm,
             dv_hbm, di_hbm,
             row_v, db0_v, db1_v, xo_v, xo_i, do_v, do_i,
             sem0, sem1):
    wid = lax.axis_index("c") * 16 + lax.axis_index("s")
    neg = jnp.full((16,), _NEG_INF, jnp.float32)
    zero = jnp.zeros((16,), jnp.int32)

    for rr in range(_XROWS_W):
        row = wid * _XROWS_W + rr
        pltpu.sync_copy(x_hbm.at[row], row_v)

        def xstep(i, st):
            v1, v2, v3, v4, t1, t2, t3, t4 = st
            cur = row_v[pl.ds(i * 16, 16)]
            ti = jnp.full((16,), i, jnp.int32)
            gt1 = cur > v1
            gt2 = cur > v2
            gt3 = cur > v3
            gt4 = cur > v4
            v4 = jnp.where(gt3, v3, jnp.where(gt4, cur, v4))
            t4 = jnp.where(gt3, t3, jnp.where(gt4, ti, t4))
            v3 = jnp.where(gt2, v2, jnp.where(gt3, cur, v3))
            t3 = jnp.where(gt2, t2, jnp.where(gt3, ti, t3))
            v2 = jnp.where(gt1, v1, jnp.where(gt2, cur, v2))
            t2 = jnp.where(gt1, t1, jnp.where(gt2, ti, t2))
            v1 = jnp.where(gt1, cur, v1)
            t1 = jnp.where(gt1, ti, t1)
            return (v1, v2, v3, v4, t1, t2, t3, t4)

        v1, v2, v3, v4, t1, t2, t3, t4 = lax.fori_loop(
            0, _XN // 16, xstep,
            (neg, neg, neg, neg, zero, zero, zero, zero), unroll=8)

        accv, acci = _merge_rounds(
            (v1, v2, v3, v4), (t1, t2, t3, t4), 4,
            jnp.zeros((16,), jnp.float32), jnp.zeros((16,), jnp.int32))
        xo_v[rr] = accv
        xo_i[rr] = acci

    xbase = wid * _XROWS_W
    pltpu.sync_copy(xo_v, xv_hbm.at[pl.ds(xbase, _XROWS_W)])
    pltpu.sync_copy(xo_i, xi_hbm.at[pl.ds(xbase, _XROWS_W)])

    dbase = wid * _DROWS_W
    bufs = (db0_v, db1_v)
    sems = (sem0, sem1)
    copies = [None, None]
    copies[0] = pltpu.async_copy(
        d_hbm.at[pl.ds(dbase, _DCH)], bufs[0], sems[0])
    for c in range(_DNCH):
        buf = bufs[c % 2]
        if c + 1 < _DNCH:
            copies[(c + 1) % 2] = pltpu.async_copy(
                d_hbm.at[pl.ds(dbase + (c + 1) * _DCH, _DCH)],
                bufs[(c + 1) % 2], sems[(c + 1) % 2])
        copies[c % 2].wait()

        def rowbody(rr, _):
            def dstep(i, st):
                v1, v2, t1, t2 = st
                cur = buf[rr, pl.ds(i * 16, 16)]
                ti = jnp.full((16,), i, jnp.int32)
                gt1 = cur > v1
                gt2 = cur > v2
                v2n = jnp.where(gt1, v1, jnp.where(gt2, cur, v2))
                t2n = jnp.where(gt1, t1, jnp.where(gt2, ti, t2))
                v1n = jnp.where(gt1, cur, v1)
                t1n = jnp.where(gt1, ti, t1)
                return (v1n, v2n, t1n, t2n)

            v1, v2, t1, t2 = lax.fori_loop(
                0, _DN // 16, dstep, (neg, neg, zero, zero), unroll=8)
            accv, acci = _merge_rounds(
                (v1, v2), (t1, t2), 2,
                jnp.zeros((16,), jnp.float32), jnp.zeros((16,), jnp.int32))
            do_v[rr] = accv
            do_i[rr] = acci
            return 0

        lax.fori_loop(0, _DCH, rowbody, 0)
        pltpu.sync_copy(do_v, dv_hbm.at[pl.ds(dbase + c * _DCH, _DCH)])
        pltpu.sync_copy(do_i, di_hbm.at[pl.ds(dbase + c * _DCH, _DCH)])


def _topk_sc(x, dflat):
    xrows = x.shape[0]
    mesh = plsc.VectorSubcoreMesh(core_axis_name="c", subcore_axis_name="s")
    kern = pl.kernel(
        _sc_body,
        out_type=[
            jax.ShapeDtypeStruct((xrows, 16), jnp.float32),
            jax.ShapeDtypeStruct((xrows, 16), jnp.int32),
            jax.ShapeDtypeStruct((_DF, 16), jnp.float32),
            jax.ShapeDtypeStruct((_DF, 16), jnp.int32),
        ],
        mesh=mesh,
        scratch_types=[
            pltpu.VMEM((_XN,), jnp.float32),
            pltpu.VMEM((_DCH, _DN), jnp.float32),
            pltpu.VMEM((_DCH, _DN), jnp.float32),
            pltpu.VMEM((_XROWS_W, 16), jnp.float32),
            pltpu.VMEM((_XROWS_W, 16), jnp.int32),
            pltpu.VMEM((_DCH, 16), jnp.float32),
            pltpu.VMEM((_DCH, 16), jnp.int32),
            pltpu.SemaphoreType.DMA,
            pltpu.SemaphoreType.DMA,
        ],
    )
    xv, xi, dv, di = kern(x, dflat)
    return xv[:, :4], xi[:, :4], dv[:, :2], di[:, :2]


_YB, _ZW, _DB = 256, 512, 8
_YS, _ZS, _DS = 16, 4, 16
_Z0, _D0 = _YS, _YS + _ZS
_STEPS = _YS + _ZS + _DS


def _topk_axis(blk, k, axis, largest=True, idx_base=0):
    idx = jax.lax.broadcasted_iota(jnp.int32, blk.shape, axis) + idx_base
    fill = _NEG_INF if largest else _POS_INF
    cur = blk
    vs, js = [], []
    for t in range(k):
        if largest:
            m = jnp.max(cur, axis=axis, keepdims=True)
        else:
            m = jnp.min(cur, axis=axis, keepdims=True)
        j = jnp.min(jnp.where(cur == m, idx, _BIG_I32), axis=axis, keepdims=True)
        vs.append(m)
        js.append(j)
        if t + 1 < k:
            cur = jnp.where(idx == j, fill, cur)
    return vs, js


def _body(y_ref, z_ref, d_ref,
          y1_ref, i1_ref, y2_ref, i2_ref,
          i3_ref, i4_ref, z1_ref, i5_ref,
          d2_ref, i8_ref,
          v_s, p_s):
    g = pl.program_id(0)

    @pl.when(g < _Z0)
    def _y_phase():
        gy = g
        blk = y_ref[...]

        (m1, m2), (j1, j2) = _topk_axis(blk, 2, 0, largest=True,
                                        idx_base=gy * _YB)

        @pl.when(gy == 0)
        def _():
            y1_ref[...] = jnp.concatenate([m1, m2], axis=0)
            i1_ref[...] = jnp.concatenate([j1, j2], axis=0)

        @pl.when(gy > 0)
        def _():
            v1 = y1_ref[0:1, :]
            v2 = y1_ref[1:2, :]
            p1 = i1_ref[0:1, :]
            p2 = i1_ref[1:2, :]
            take1 = m1 > v1
            a_v = jnp.where(take1, v1, v2)
            a_i = jnp.where(take1, p1, p2)
            b_v = jnp.where(take1, m2, m1)
            b_i = jnp.where(take1, j2, j1)
            take2 = b_v > a_v
            y1_ref[...] = jnp.concatenate(
                [jnp.where(take1, m1, v1), jnp.where(take2, b_v, a_v)], axis=0)
            i1_ref[...] = jnp.concatenate(
                [jnp.where(take1, j1, p1), jnp.where(take2, b_i, a_i)], axis=0)

        (n1, n2), (k1, k2) = _topk_axis(blk, 2, 1, largest=False)
        y2_ref[...] = jnp.concatenate([n1, n2], axis=1)
        i2_ref[...] = jnp.concatenate([k1, k2], axis=1)

    @pl.when(jnp.logical_and(g >= _Z0, g < _D0))
    def _z_phase():
        gz = g - _Z0
        blk = z_ref[...]

        _, js = _topk_axis(blk, 2, 0, largest=True)
        i3_ref[...] = jnp.concatenate(js, axis=0)

        _, js = _topk_axis(blk, 3, 1, largest=True)
        i4_ref[...] = jnp.concatenate(js, axis=1)

        lanes = jax.lax.broadcasted_iota(jnp.int32, blk.shape, 2) + gz * _ZW
        lm = jnp.max(blk, axis=2)
        lj = jnp.min(jnp.where(blk == lm[:, :, None], lanes, _BIG_I32), axis=2)

        @pl.when(gz == 0)
        def _():
            v_s[...] = lm
            p_s[...] = lj

        @pl.when(gz > 0)
        def _():
            take = lm > v_s[...]
            v_s[...] = jnp.where(take, lm, v_s[...])
            p_s[...] = jnp.where(take, lj, p_s[...])

        @pl.when(gz == _ZS - 1)
        def _():
            z1_ref[...] = v_s[...]
            i5_ref[...] = p_s[...]

    @pl.when(g >= _D0)
    def _d_phase():
        blk = d_ref[...]

        vs, js = _topk_axis(blk, 2, 1, largest=True)
        d2_ref[...] = jnp.concatenate(vs, axis=1)
        i8_ref[...] = jnp.concatenate(js, axis=1)


def _fused(y, z, d):
    yr, yn = y.shape
    a0, a1, a2 = z.shape
    db, d1, d2n = d.shape

    def ym(g):
        return (jnp.clip(g, 0, _YS - 1), 0)

    def zm(g):
        return (0, 0, jnp.clip(g - _Z0, 0, _ZS - 1))

    def dm(g):
        return (jnp.clip(g - _D0, 0, _DS - 1), 0, 0)

    return pl.pallas_call(
        _body,
        grid=(_STEPS,),
        in_specs=[
            pl.BlockSpec((_YB, yn), ym),
            pl.BlockSpec((a0, a1, _ZW), zm),
            pl.BlockSpec((_DB, d1, d2n), dm),
        ],
        out_specs=[
            pl.BlockSpec((2, yn), lambda g: (0, 0)),
            pl.BlockSpec((2, yn), lambda g: (0, 0)),
            pl.BlockSpec((_YB, 2), ym),
            pl.BlockSpec((_YB, 2), ym),
            pl.BlockSpec((2, a1, _ZW), zm),
            pl.BlockSpec((a0, 3, _ZW), zm),
            pl.BlockSpec((a0, a1), lambda g: (0, 0)),
            pl.BlockSpec((a0, a1), lambda g: (0, 0)),
            pl.BlockSpec((_DB, 2, d2n), dm),
            pl.BlockSpec((_DB, 2, d2n), dm),
        ],
        out_shape=[
            jax.ShapeDtypeStruct((2, yn), jnp.float32),
            jax.ShapeDtypeStruct((2, yn), jnp.int32),
            jax.ShapeDtypeStruct((yr, 2), jnp.float32),
            jax.ShapeDtypeStruct((yr, 2), jnp.int32),
            jax.ShapeDtypeStruct((2, a1, a2), jnp.int32),
            jax.ShapeDtypeStruct((a0, 3, a2), jnp.int32),
            jax.ShapeDtypeStruct((a0, a1), jnp.float32),
            jax.ShapeDtypeStruct((a0, a1), jnp.int32),
            jax.ShapeDtypeStruct((db, 2, d2n), jnp.float32),
            jax.ShapeDtypeStruct((db, 2, d2n), jnp.int32),
        ],
        scratch_shapes=[
            pltpu.VMEM((a0, a1), jnp.float32),
            pltpu.VMEM((a0, a1), jnp.int32),
        ],
    )(y, z, d)


def kernel(x, y, z, d):
    b0, b1, a1, a2 = d.shape
    x0, i0, d3, i9 = _topk_sc(x, d.reshape(b0 * b1 * a1, a2))

    (y1, i1, y2, i2, i3, i4, z1, i5,
     d2, i8) = _fused(y, z, d.reshape(b0 * b1, a1, a2))
    z1 = z1[:, :, None]
    i5 = i5[:, :, None]
    d2 = d2.reshape(b0, b1, 2, a2)
    i8 = i8.reshape(b0, b1, 2, a2)
    d3 = d3.reshape(b0, b1, a1, 2)
    i9 = i9.reshape(b0, b1, a1, 2)
    return (x0, y1, y2, i0, i1, i2, z1, i3, i4, i5, d2, d3, i8, i9)
